# Initial kernel scaffold; baseline (speedup 1.0000x reference)
import jax
import jax.numpy as jnp
from jax.experimental import pallas as pl


def kernel(x_paper, x_author, edge_index_writes, edge_index_written_by, edge_index_cites, params):
    raise NotImplementedError("write your pallas kernel here")


# speedup vs baseline: 28.9478x; 28.9478x over previous
import functools

import jax
import jax.numpy as jnp
import numpy as np
from jax import lax
from jax.experimental import pallas as pl
from jax.experimental.pallas import tpu as pltpu
from jax.experimental.pallas import tpu_sc as plsc

C = 128
H = 2
D = 64
NP = 10000
NA = 10000
NPAD = 10240
E = 160000
_NT = 16
_EPT = E // _NT
_B = 80
_NBATCH = _EPT // _B
_RPT = 2 * NPAD // _NT
_ZC = 80
_BN = 640
_INV_SQRT2 = 0.7071067811865476
_ISD = 1.0 / 8.0


def _proj_body(x_ref, wq, bq, wk, bk, wv, bv, ar, mr, q_out, k_out, v_out):
    x = x_ref[...]
    qq = jnp.dot(x, wq[0], preferred_element_type=jnp.float32) + bq[0]
    q_out[0, :, :] = qq[:, :D]
    q_out[1, :, :] = qq[:, D:]
    kk = jnp.dot(x, wk[0], preferred_element_type=jnp.float32) + bk[0]
    vv = jnp.dot(x, wv[0], preferred_element_type=jnp.float32) + bv[0]
    k_out[0, :, :] = jnp.dot(kk[:, :D], ar[0, 0], preferred_element_type=jnp.float32)
    k_out[1, :, :] = jnp.dot(kk[:, D:], ar[0, 1], preferred_element_type=jnp.float32)
    v_out[0, :, :] = jnp.dot(vv[:, :D], mr[0, 0], preferred_element_type=jnp.float32)
    v_out[1, :, :] = jnp.dot(vv[:, D:], mr[0, 1], preferred_element_type=jnp.float32)


def _proj(x_all, wq_st, bq_st, wk_st, bk_st, wv_st, bv_st, arel_st, mrel_st):
    nb = NPAD // _BN

    def _xrow(i):
        return jnp.where(i < 2 * nb, i % nb, nb + i % nb)

    def _wsel(i):
        return jnp.where(i < 2 * nb, 0, 1)

    w_spec = pl.BlockSpec((1, C, C), lambda i: (_wsel(i), 0, 0))
    b_spec = pl.BlockSpec((1, 1, C), lambda i: (_wsel(i), 0, 0))
    e_spec = pl.BlockSpec((1, H, D, D), lambda i: (i // nb, 0, 0, 0))
    f = pl.pallas_call(
        _proj_body,
        grid=(3 * nb,),
        in_specs=[pl.BlockSpec((_BN, C), lambda i: (_xrow(i), 0)),
                  w_spec, b_spec, w_spec, b_spec, w_spec, b_spec,
                  e_spec, e_spec],
        out_specs=[
            pl.BlockSpec((H, _BN, D),
                         lambda i: (0, ((i // nb + 1) % 3) * nb + i % nb, 0)),
            pl.BlockSpec((H, _BN, D), lambda i: (0, i, 0)),
            pl.BlockSpec((H, _BN, D), lambda i: (0, i, 0)),
        ],
        out_shape=[jax.ShapeDtypeStruct((H, 3 * NPAD, D), jnp.float32)] * 3,
    )
    return f(x_all, wq_st, bq_st, wk_st, bk_st, wv_st, bv_st, arel_st, mrel_st)


def _out_body(o_ref, x_ref, wa, ba, gam, out_ref):
    o0 = o_ref[0, :, :]
    o1 = o_ref[1, :, :]
    g0 = 0.5 * o0 * (1.0 + lax.erf(o0 * _INV_SQRT2))
    g1 = 0.5 * o1 * (1.0 + lax.erf(o1 * _INV_SQRT2))
    y = (jnp.dot(g0, wa[0, :D, :], preferred_element_type=jnp.float32)
         + jnp.dot(g1, wa[0, D:, :], preferred_element_type=jnp.float32)
         + ba[0])
    out_ref[...] = y + gam[0] * x_ref[...]


def _outproj(o_all, x_all, wa_st, ba_st, gam_st):
    nb = NPAD // _BN
    f = pl.pallas_call(
        _out_body,
        grid=(2 * nb,),
        in_specs=[pl.BlockSpec((H, _BN, D), lambda i: (0, i, 0)),
                  pl.BlockSpec((_BN, C), lambda i: (i, 0)),
                  pl.BlockSpec((1, C, C), lambda i: (i // nb, 0, 0)),
                  pl.BlockSpec((1, 1, C), lambda i: (i // nb, 0, 0)),
                  pl.BlockSpec((1, 1, C), lambda i: (i // nb, 0, 0))],
        out_specs=pl.BlockSpec((_BN, C), lambda i: (i, 0)),
        out_shape=jax.ShapeDtypeStruct((2 * NPAD, C), jnp.float32),
    )
    return f(o_all, x_all, wa_st, ba_st, gam_st)


def _sc_body(src_all, dst_all, qtab, ktab, vtab,
             o_out,
             sidx_v, didx_v, doff_v, q_rows, k_rows, v_rows, ex_v, zb1,
             num_s, den_s, sem):
    h = lax.axis_index("c")
    tid = lax.axis_index("s")
    r0 = tid * _RPT
    zv = jnp.zeros((16,), jnp.float32)
    lidx = lax.iota(jnp.int32, 16)

    def _zrow(i, _):
        for t in range(4):
            k_rows[i, pl.ds(16 * t, 16)] = zv
        return 0
    lax.fori_loop(0, _ZC, _zrow, 0)

    def _zb(j, _):
        zb1[pl.ds(j * 16, 16)] = zv
        return 0
    lax.fori_loop(0, _RPT // 16, _zb, 0)

    def _zc(cc, _):
        pltpu.sync_copy(k_rows, num_s.at[pl.ds(r0 + cc * _ZC, _ZC)])
        return 0
    lax.fori_loop(0, _RPT // _ZC, _zc, 0)
    pltpu.sync_copy(zb1, den_s.at[pl.ds(r0, _RPT)])
    plsc.subcore_barrier()

    def _batch(b, _):
        et = b // _NBATCH
        lb = b - et * _NBATCH
        slot = jnp.where(et == 0, 2, et - 1)
        tbase = h * (3 * NPAD) + slot * NPAD
        accoff = jnp.where(et == 1, NPAD, 0)
        base = et * E + tid * _EPT + lb * _B
        pltpu.sync_copy(src_all.at[pl.ds(base, _B)], sidx_v)
        pltpu.sync_copy(dst_all.at[pl.ds(base, _B)], didx_v)

        def _off(j, _2):
            doff_v[pl.ds(j * 16, 16)] = didx_v[pl.ds(j * 16, 16)] + tbase
            didx_v[pl.ds(j * 16, 16)] = didx_v[pl.ds(j * 16, 16)] + accoff
            sidx_v[pl.ds(j * 16, 16)] = sidx_v[pl.ds(j * 16, 16)] + tbase
            return 0
        lax.fori_loop(0, _B // 16, _off, 0)

        cq = pltpu.async_copy(qtab.at[doff_v], q_rows, sem)
        ck = pltpu.async_copy(ktab.at[sidx_v], k_rows, sem)
        cv = pltpu.async_copy(vtab.at[sidx_v], v_rows, sem)
        cq.wait()
        ck.wait()
        cv.wait()

        def _grp(g, _2):
            accv = jnp.zeros((16,), jnp.float32)
            exvs = []
            for u in range(16):
                i = g * 16 + u
                acc = q_rows[i, pl.ds(0, 16)] * k_rows[i, pl.ds(0, 16)]
                for t in range(1, 4):
                    acc = acc + q_rows[i, pl.ds(16 * t, 16)] * k_rows[i, pl.ds(16 * t, 16)]
                s = jnp.sum(acc)
                exv = jnp.exp(lax.broadcast_in_dim(s, (16,), ()))
                exvs.append(exv)
                accv = jnp.where(lidx == u, exv, accv)
            ex_v[pl.ds(g * 16, 16)] = accv
            for u in range(16):
                i = g * 16 + u
                for t in range(4):
                    v_rows[i, pl.ds(16 * t, 16)] = v_rows[i, pl.ds(16 * t, 16)] * exvs[u]
            return 0
        lax.fori_loop(0, _B // 16, _grp, 0)

        pltpu.sync_copy(v_rows, num_s.at[didx_v], add=True)
        pltpu.sync_copy(ex_v, den_s.at[didx_v], add=True)
        return 0
    lax.fori_loop(0, 3 * _NBATCH, _batch, 0)
    plsc.subcore_barrier()

    def _chunk(cc, _):
        rr = r0 + cc * _ZC
        pltpu.sync_copy(num_s.at[pl.ds(rr, _ZC)], q_rows)
        pltpu.sync_copy(den_s.at[pl.ds(rr, _ZC)], ex_v)

        def _div(g, _2):
            dv16 = ex_v[pl.ds(g * 16, 16)]
            for u in range(16):
                i = g * 16 + u
                ds = jnp.sum(jnp.where(lidx == u, dv16, 0.0))
                dv = lax.broadcast_in_dim(ds, (16,), ()) + 1e-16
                for t in range(4):
                    q_rows[i, pl.ds(16 * t, 16)] = q_rows[i, pl.ds(16 * t, 16)] / dv
            return 0
        lax.fori_loop(0, _ZC // 16, _div, 0)
        pltpu.sync_copy(q_rows, o_out.at[h, pl.ds(rr, _ZC)])
        return 0
    lax.fori_loop(0, _RPT // _ZC, _chunk, 0)


_sc_edge = functools.partial(
    pl.kernel,
    out_type=jax.ShapeDtypeStruct((H, 2 * NPAD, D), jnp.float32),
    mesh=plsc.VectorSubcoreMesh(core_axis_name="c", subcore_axis_name="s"),
    compiler_params=pltpu.CompilerParams(needs_layout_passes=False,
                                         use_tc_tiling_on_sc=False),
    scratch_types=[
        pltpu.VMEM((_B,), jnp.int32),
        pltpu.VMEM((_B,), jnp.int32),
        pltpu.VMEM((_B,), jnp.int32),
        pltpu.VMEM((_B, D), jnp.float32),
        pltpu.VMEM((_B, D), jnp.float32),
        pltpu.VMEM((_B, D), jnp.float32),
        pltpu.VMEM((_B,), jnp.float32),
        pltpu.VMEM((_RPT,), jnp.float32),
        pltpu.VMEM_SHARED((2 * NPAD, D), jnp.float32),
        pltpu.VMEM_SHARED((2 * NPAD,), jnp.float32),
        pltpu.SemaphoreType.DMA,
    ],
)(_sc_body)


def kernel(x_paper, x_author, edge_index_writes, edge_index_written_by,
           edge_index_cites, params):
    src_all = jnp.concatenate([
        edge_index_writes[0], edge_index_written_by[0], edge_index_cites[0],
    ]).astype(jnp.int32)
    dst_all = jnp.concatenate([
        edge_index_writes[1], edge_index_written_by[1], edge_index_cites[1],
    ]).astype(jnp.int32)

    zpad = jnp.zeros((NPAD - NP, C), jnp.float32)
    x_all = jnp.concatenate([x_paper, zpad, x_author, zpad], axis=0)

    for l in range(2):
        nd = params[l]["node"]
        ed = params[l]["edge"]
        pw = ed["author__writes__paper"]
        pwb = ed["paper__written_by__author"]
        pc = ed["paper__cites__paper"]
        pp = nd["paper"]
        pa = nd["author"]

        def _ar(pe):
            return pe["a_rel"] * (pe["p_rel"] * _ISD)[:, None, None]

        wq_st = jnp.stack([pp["Wq"], pa["Wq"]])
        bq_st = jnp.stack([pp["bq"].reshape(1, C), pa["bq"].reshape(1, C)])
        wk_st = jnp.stack([pp["Wk"], pa["Wk"]])
        bk_st = jnp.stack([pp["bk"].reshape(1, C), pa["bk"].reshape(1, C)])
        wv_st = jnp.stack([pp["Wv"], pa["Wv"]])
        bv_st = jnp.stack([pp["bv"].reshape(1, C), pa["bv"].reshape(1, C)])
        arel_st = jnp.stack([_ar(pwb), _ar(pc), _ar(pw)])
        mrel_st = jnp.stack([pwb["m_rel"], pc["m_rel"], pw["m_rel"]])

        q_all, ktab, vtab = _proj(x_all, wq_st, bq_st, wk_st, bk_st,
                                  wv_st, bv_st, arel_st, mrel_st)

        o_all = _sc_edge(src_all, dst_all,
                         q_all.reshape(H * 3 * NPAD, D),
                         ktab.reshape(H * 3 * NPAD, D),
                         vtab.reshape(H * 3 * NPAD, D))

        bp = jax.nn.sigmoid(pp["skip"])
        ba = jax.nn.sigmoid(pa["skip"])
        ones_row = jnp.ones((1, C), jnp.float32)
        wa_st = jnp.stack([pp["Wa"] * bp, pa["Wa"] * ba])
        ba_st = jnp.stack([(pp["ba"] * bp).reshape(1, C),
                           (pa["ba"] * ba).reshape(1, C)])
        gam_st = jnp.stack([(1.0 - bp) * ones_row, (1.0 - ba) * ones_row])
        x_all = _outproj(o_all, x_all, wa_st, ba_st, gam_st)

    x_p = lax.slice(x_all, (0, 0), (NP, C))
    x_a = lax.slice(x_all, (NPAD, 0), (NPAD + NA, C))
    return (x_p, x_a)


# speedup vs baseline: 35.3059x; 1.2196x over previous
import functools

import jax
import jax.numpy as jnp
import numpy as np
from jax import lax
from jax.experimental import pallas as pl
from jax.experimental.pallas import tpu as pltpu
from jax.experimental.pallas import tpu_sc as plsc

C = 128
H = 2
D = 64
NP = 10000
NA = 10000
NPAD = 10240
E = 160000
_NT = 16
_EPT = E // _NT
_B = 80
_NBATCH = _EPT // _B
_RPT = 2 * NPAD // _NT
_ZC = 80
_BN = 640
_INV_SQRT2 = 0.7071067811865476
_ISD = 1.0 / 8.0


def _proj_body(x_ref, wq, bq, wk, bk, wv, bv, ar, mr, q_out, k_out, v_out):
    x = x_ref[...]
    qq = jnp.dot(x, wq[0], preferred_element_type=jnp.float32) + bq[0]
    q_out[0, :, :] = qq[:, :D]
    q_out[1, :, :] = qq[:, D:]
    kk = jnp.dot(x, wk[0], preferred_element_type=jnp.float32) + bk[0]
    vv = jnp.dot(x, wv[0], preferred_element_type=jnp.float32) + bv[0]
    k_out[0, :, :] = jnp.dot(kk[:, :D], ar[0, 0], preferred_element_type=jnp.float32)
    k_out[1, :, :] = jnp.dot(kk[:, D:], ar[0, 1], preferred_element_type=jnp.float32)
    v_out[0, :, :] = jnp.dot(vv[:, :D], mr[0, 0], preferred_element_type=jnp.float32)
    v_out[1, :, :] = jnp.dot(vv[:, D:], mr[0, 1], preferred_element_type=jnp.float32)


def _proj(x_all, wq_st, bq_st, wk_st, bk_st, wv_st, bv_st, arel_st, mrel_st):
    nb = NPAD // _BN

    def _xrow(i):
        return jnp.where(i < 2 * nb, i % nb, nb + i % nb)

    def _wsel(i):
        return jnp.where(i < 2 * nb, 0, 1)

    w_spec = pl.BlockSpec((1, C, C), lambda i: (_wsel(i), 0, 0))
    b_spec = pl.BlockSpec((1, 1, C), lambda i: (_wsel(i), 0, 0))
    e_spec = pl.BlockSpec((1, H, D, D), lambda i: (i // nb, 0, 0, 0))
    f = pl.pallas_call(
        _proj_body,
        grid=(3 * nb,),
        in_specs=[pl.BlockSpec((_BN, C), lambda i: (_xrow(i), 0)),
                  w_spec, b_spec, w_spec, b_spec, w_spec, b_spec,
                  e_spec, e_spec],
        out_specs=[
            pl.BlockSpec((H, _BN, D),
                         lambda i: (0, ((i // nb + 1) % 3) * nb + i % nb, 0)),
            pl.BlockSpec((H, _BN, D), lambda i: (0, i, 0)),
            pl.BlockSpec((H, _BN, D), lambda i: (0, i, 0)),
        ],
        out_shape=[jax.ShapeDtypeStruct((H, 3 * NPAD, D), jnp.float32)] * 3,
    )
    return f(x_all, wq_st, bq_st, wk_st, bk_st, wv_st, bv_st, arel_st, mrel_st)


def _out_body(o_ref, x_ref, wa, ba, gam, out_ref):
    o0 = o_ref[0, :, :]
    o1 = o_ref[1, :, :]
    g0 = 0.5 * o0 * (1.0 + lax.erf(o0 * _INV_SQRT2))
    g1 = 0.5 * o1 * (1.0 + lax.erf(o1 * _INV_SQRT2))
    y = (jnp.dot(g0, wa[0, :D, :], preferred_element_type=jnp.float32)
         + jnp.dot(g1, wa[0, D:, :], preferred_element_type=jnp.float32)
         + ba[0])
    out_ref[...] = y + gam[0] * x_ref[...]


def _outproj(o_all, x_all, wa_st, ba_st, gam_st):
    nb = NPAD // _BN
    f = pl.pallas_call(
        _out_body,
        grid=(2 * nb,),
        in_specs=[pl.BlockSpec((H, _BN, D), lambda i: (0, i, 0)),
                  pl.BlockSpec((_BN, C), lambda i: (i, 0)),
                  pl.BlockSpec((1, C, C), lambda i: (i // nb, 0, 0)),
                  pl.BlockSpec((1, 1, C), lambda i: (i // nb, 0, 0)),
                  pl.BlockSpec((1, 1, C), lambda i: (i // nb, 0, 0))],
        out_specs=pl.BlockSpec((_BN, C), lambda i: (i, 0)),
        out_shape=jax.ShapeDtypeStruct((2 * NPAD, C), jnp.float32),
    )
    return f(o_all, x_all, wa_st, ba_st, gam_st)


def _sc_body(src_all, dst_all, qtab, ktab, vtab,
             o_out,
             sidx_a, didx_a, doff_a, q_a, k_a, v_a,
             sidx_b, didx_b, doff_b, q_b, k_b, v_b,
             ex_v, zb1, num_s, den_s, sem_a, sem_b):
    h = lax.axis_index("c")
    tid = lax.axis_index("s")
    r0 = tid * _RPT
    zv = jnp.zeros((16,), jnp.float32)
    lidx = lax.iota(jnp.int32, 16)
    bufs_a = (sidx_a, didx_a, doff_a, q_a, k_a, v_a, sem_a)
    bufs_b = (sidx_b, didx_b, doff_b, q_b, k_b, v_b, sem_b)

    def _zrow(i, _):
        for t in range(4):
            k_a[i, pl.ds(16 * t, 16)] = zv
        return 0
    lax.fori_loop(0, _ZC, _zrow, 0)

    def _zb(j, _):
        zb1[pl.ds(j * 16, 16)] = zv
        return 0
    lax.fori_loop(0, _RPT // 16, _zb, 0)

    def _zc(cc, _):
        pltpu.sync_copy(k_a, num_s.at[pl.ds(r0 + cc * _ZC, _ZC)])
        return 0
    lax.fori_loop(0, _RPT // _ZC, _zc, 0)
    pltpu.sync_copy(zb1, den_s.at[pl.ds(r0, _RPT)])
    plsc.subcore_barrier()

    def _issue(b, bufs):
        sidx_v, didx_v, doff_v, q_rows, k_rows, v_rows, sem = bufs
        et = b // _NBATCH
        lb = b - et * _NBATCH
        slot = jnp.where(et == 0, 2, et - 1)
        tbase = h * (3 * NPAD) + slot * NPAD
        accoff = jnp.where(et == 1, NPAD, 0)
        base = et * E + tid * _EPT + lb * _B
        pltpu.sync_copy(src_all.at[pl.ds(base, _B)], sidx_v)
        pltpu.sync_copy(dst_all.at[pl.ds(base, _B)], didx_v)

        def _off(j, _2):
            doff_v[pl.ds(j * 16, 16)] = didx_v[pl.ds(j * 16, 16)] + tbase
            didx_v[pl.ds(j * 16, 16)] = didx_v[pl.ds(j * 16, 16)] + accoff
            sidx_v[pl.ds(j * 16, 16)] = sidx_v[pl.ds(j * 16, 16)] + tbase
            return 0
        lax.fori_loop(0, _B // 16, _off, 0)
        pltpu.async_copy(qtab.at[doff_v], q_rows, sem)
        pltpu.async_copy(ktab.at[sidx_v], k_rows, sem)
        pltpu.async_copy(vtab.at[sidx_v], v_rows, sem)

    def _work(bufs):
        sidx_v, didx_v, doff_v, q_rows, k_rows, v_rows, sem = bufs
        pltpu.make_async_copy(qtab.at[doff_v], q_rows, sem).wait()
        pltpu.make_async_copy(ktab.at[sidx_v], k_rows, sem).wait()
        pltpu.make_async_copy(vtab.at[sidx_v], v_rows, sem).wait()

        def _grp(g, _2):
            accv = jnp.zeros((16,), jnp.float32)
            exvs = []
            for u in range(16):
                i = g * 16 + u
                acc = q_rows[i, pl.ds(0, 16)] * k_rows[i, pl.ds(0, 16)]
                for t in range(1, 4):
                    acc = acc + q_rows[i, pl.ds(16 * t, 16)] * k_rows[i, pl.ds(16 * t, 16)]
                s = jnp.sum(acc)
                exv = jnp.exp(lax.broadcast_in_dim(s, (16,), ()))
                exvs.append(exv)
                accv = jnp.where(lidx == u, exv, accv)
            ex_v[pl.ds(g * 16, 16)] = accv
            for u in range(16):
                i = g * 16 + u
                for t in range(4):
                    v_rows[i, pl.ds(16 * t, 16)] = v_rows[i, pl.ds(16 * t, 16)] * exvs[u]
            return 0
        lax.fori_loop(0, _B // 16, _grp, 0)
        pltpu.sync_copy(v_rows, num_s.at[didx_v], add=True)
        pltpu.sync_copy(ex_v, den_s.at[didx_v], add=True)

    ntot = 3 * _NBATCH
    _issue(jnp.int32(0), bufs_a)

    def _pair(g, _):
        _issue(2 * g + 1, bufs_b)
        _work(bufs_a)
        _issue(2 * g + 2, bufs_a)
        _work(bufs_b)
        return 0
    lax.fori_loop(0, ntot // 2, _pair, 0)
    _work(bufs_a)
    plsc.subcore_barrier()

    def _chunk(cc, _):
        rr = r0 + cc * _ZC
        pltpu.sync_copy(num_s.at[pl.ds(rr, _ZC)], q_a)
        pltpu.sync_copy(den_s.at[pl.ds(rr, _ZC)], ex_v)

        def _div(g, _2):
            dv16 = ex_v[pl.ds(g * 16, 16)]
            for u in range(16):
                i = g * 16 + u
                ds = jnp.sum(jnp.where(lidx == u, dv16, 0.0))
                dv = lax.broadcast_in_dim(ds, (16,), ()) + 1e-16
                for t in range(4):
                    q_a[i, pl.ds(16 * t, 16)] = q_a[i, pl.ds(16 * t, 16)] / dv
            return 0
        lax.fori_loop(0, _ZC // 16, _div, 0)
        pltpu.sync_copy(q_a, o_out.at[h, pl.ds(rr, _ZC)])
        return 0
    lax.fori_loop(0, _RPT // _ZC, _chunk, 0)


_sc_edge = functools.partial(
    pl.kernel,
    out_type=jax.ShapeDtypeStruct((H, 2 * NPAD, D), jnp.float32),
    mesh=plsc.VectorSubcoreMesh(core_axis_name="c", subcore_axis_name="s"),
    compiler_params=pltpu.CompilerParams(needs_layout_passes=False,
                                         use_tc_tiling_on_sc=False),
    scratch_types=(
        [pltpu.VMEM((_B,), jnp.int32)] * 3
        + [pltpu.VMEM((_B, D), jnp.float32)] * 3
        + [pltpu.VMEM((_B,), jnp.int32)] * 3
        + [pltpu.VMEM((_B, D), jnp.float32)] * 3
        + [
            pltpu.VMEM((_B,), jnp.float32),
            pltpu.VMEM((_RPT,), jnp.float32),
            pltpu.VMEM_SHARED((2 * NPAD, D), jnp.float32),
            pltpu.VMEM_SHARED((2 * NPAD,), jnp.float32),
            pltpu.SemaphoreType.DMA,
            pltpu.SemaphoreType.DMA,
        ]
    ),
)(_sc_body)


def kernel(x_paper, x_author, edge_index_writes, edge_index_written_by,
           edge_index_cites, params):
    src_all = jnp.concatenate([
        edge_index_writes[0], edge_index_written_by[0], edge_index_cites[0],
    ]).astype(jnp.int32)
    dst_all = jnp.concatenate([
        edge_index_writes[1], edge_index_written_by[1], edge_index_cites[1],
    ]).astype(jnp.int32)

    zpad = jnp.zeros((NPAD - NP, C), jnp.float32)
    x_all = jnp.concatenate([x_paper, zpad, x_author, zpad], axis=0)

    for l in range(2):
        nd = params[l]["node"]
        ed = params[l]["edge"]
        pw = ed["author__writes__paper"]
        pwb = ed["paper__written_by__author"]
        pc = ed["paper__cites__paper"]
        pp = nd["paper"]
        pa = nd["author"]

        def _ar(pe):
            return pe["a_rel"] * (pe["p_rel"] * _ISD)[:, None, None]

        wq_st = jnp.stack([pp["Wq"], pa["Wq"]])
        bq_st = jnp.stack([pp["bq"].reshape(1, C), pa["bq"].reshape(1, C)])
        wk_st = jnp.stack([pp["Wk"], pa["Wk"]])
        bk_st = jnp.stack([pp["bk"].reshape(1, C), pa["bk"].reshape(1, C)])
        wv_st = jnp.stack([pp["Wv"], pa["Wv"]])
        bv_st = jnp.stack([pp["bv"].reshape(1, C), pa["bv"].reshape(1, C)])
        arel_st = jnp.stack([_ar(pwb), _ar(pc), _ar(pw)])
        mrel_st = jnp.stack([pwb["m_rel"], pc["m_rel"], pw["m_rel"]])

        q_all, ktab, vtab = _proj(x_all, wq_st, bq_st, wk_st, bk_st,
                                  wv_st, bv_st, arel_st, mrel_st)

        o_all = _sc_edge(src_all, dst_all,
                         q_all.reshape(H * 3 * NPAD, D),
                         ktab.reshape(H * 3 * NPAD, D),
                         vtab.reshape(H * 3 * NPAD, D))

        bp = jax.nn.sigmoid(pp["skip"])
        ba = jax.nn.sigmoid(pa["skip"])
        ones_row = jnp.ones((1, C), jnp.float32)
        wa_st = jnp.stack([pp["Wa"] * bp, pa["Wa"] * ba])
        ba_st = jnp.stack([(pp["ba"] * bp).reshape(1, C),
                           (pa["ba"] * ba).reshape(1, C)])
        gam_st = jnp.stack([(1.0 - bp) * ones_row, (1.0 - ba) * ones_row])
        x_all = _outproj(o_all, x_all, wa_st, ba_st, gam_st)

    x_p = lax.slice(x_all, (0, 0), (NP, C))
    x_a = lax.slice(x_all, (NPAD, 0), (NPAD + NA, C))
    return (x_p, x_a)
